# baseline (speedup 1.0000x reference)
import functools

import jax
import jax.numpy as jnp
from jax import lax
from jax.experimental import pallas as pl
from jax.experimental.pallas import tpu as pltpu
from jax.experimental.pallas import tpu_sc as plsc

NTILES = 32
NGRP = 2
CGRP = 16
CH = 256
LANES = 16


def _sc_maxdiff(xw, idxb, n_tab, n_pad, k_deg):
    pairs = xw.shape[1] // n_tab
    n_sub = n_pad // NGRP
    blocks = n_sub // CH
    groups = CH // LANES
    mesh = plsc.VectorSubcoreMesh(core_axis_name="c", subcore_axis_name="s")

    @functools.partial(
        pl.kernel,
        out_type=jax.ShapeDtypeStruct((2 * CGRP * pairs, n_pad), jnp.float32),
        mesh=mesh,
        compiler_params=pltpu.CompilerParams(
            needs_layout_passes=False,
            use_tc_tiling_on_sc=False,
        ),
        scratch_types=[
            pltpu.VMEM((pairs * n_tab,), jnp.int32),
            pltpu.VMEM((2 * pairs, n_sub), jnp.float32),
            pltpu.VMEM((k_deg, CH), jnp.int32),
            pltpu.VMEM((k_deg, CH), jnp.int32),
            pltpu.VMEM((k_deg, CH), jnp.int32),
            pltpu.VMEM((k_deg, CH), jnp.int32),
            pltpu.SemaphoreType.DMA,
            pltpu.SemaphoreType.DMA,
        ],
    )
    def sc_kernel(x_hbm, idx_hbm, m_hbm, table_v, out_v,
                  i0a, i1a, i0b, i1b, sem_a, sem_b):
        wid = lax.axis_index("s") * 2 + lax.axis_index("c")
        cg = wid // NGRP
        ng = wid % NGRP
        poffs = [jnp.full((LANES,), p * n_tab, jnp.int32) for p in range(pairs)]
        blk0 = ng * blocks

        def issue(blk, d0, d1, sem):
            pltpu.async_copy(idx_hbm.at[0, blk], d0, sem)
            pltpu.async_copy(idx_hbm.at[1, blk], d1, sem)

        def drain(blk, d0, d1, sem):
            pltpu.make_async_copy(idx_hbm.at[0, blk], d0, sem).wait()
            pltpu.make_async_copy(idx_hbm.at[1, blk], d1, sem).wait()

        def compute(j, b0, b1):
            def g_body(g, carry):
                base = g * LANES
                accs = []
                for kk in range(k_deg):
                    i0 = b0[kk, pl.ds(base, LANES)]
                    i1 = b1[kk, pl.ds(base, LANES)]
                    for p in range(pairs):
                        a = plsc.load_gather(table_v, [i0 + poffs[p]])
                        b2 = plsc.load_gather(table_v, [i1 + poffs[p]])
                        d = plsc.bitcast(a, jnp.bfloat16) - plsc.bitcast(b2, jnp.bfloat16)
                        if kk == 0:
                            accs.append(d)
                        else:
                            accs[p] = jnp.maximum(accs[p], d)
                loc = j * CH + base
                for p in range(pairs):
                    lo, hi = plsc.unpack(accs[p], format=plsc.PackFormat.INTERLEAVED)
                    out_v[2 * p, pl.ds(loc, LANES)] = lo
                    out_v[2 * p + 1, pl.ds(loc, LANES)] = hi
                return carry

            lax.fori_loop(0, groups, g_body, 0)

        issue(blk0, i0a, i1a, sem_a)
        pltpu.sync_copy(x_hbm.at[cg], table_v)

        def super_body(it, carry):
            ja = 2 * it
            jb = 2 * it + 1
            issue(blk0 + jb, i0b, i1b, sem_b)
            drain(blk0 + ja, i0a, i1a, sem_a)
            compute(ja, i0a, i1a)

            @pl.when(it + 1 < blocks // 2)
            def _():
                issue(blk0 + ja + 2, i0a, i1a, sem_a)

            drain(blk0 + jb, i0b, i1b, sem_b)
            compute(jb, i0b, i1b)
            return carry

        lax.fori_loop(0, blocks // 2, super_body, 0)
        pltpu.sync_copy(
            out_v,
            m_hbm.at[pl.ds(cg * 2 * pairs, 2 * pairs), pl.ds(ng * n_sub, n_sub)],
        )

    return sc_kernel(xw, idxb)


def _tc_partial(xf, We, b2, c, n):
    def body(x_ref, we_ref, b_ref, y_ref):
        y_ref[...] = (
            jnp.dot(we_ref[...], x_ref[...], preferred_element_type=jnp.float32)
            + b_ref[...]
        )

    return pl.pallas_call(
        body,
        out_shape=jax.ShapeDtypeStruct((c, n), jnp.float32),
    )(xf, We, b2)


def _tc_conv(part, m, Wo, c, n):
    def body(p_ref, m_ref, wo_ref, y_ref):
        acc = p_ref[...] + jnp.dot(
            wo_ref[...], m_ref[:, :n], preferred_element_type=jnp.float32
        )
        y_ref[...] = jnp.maximum(acc, 0.0)

    return pl.pallas_call(
        body,
        out_shape=jax.ShapeDtypeStruct((c, n), jnp.float32),
    )(part, m, Wo)


def kernel(x, edge_index, W, b):
    B, C, N, _ = x.shape
    K = edge_index.shape[-1]
    n_pad = ((N + (NGRP * CH) - 1) // (NGRP * CH)) * (NGRP * CH)
    nblk = n_pad // CH
    pairs = C // (2 * CGRP)

    xf = x.reshape(C, N)
    xT = jnp.transpose(x[0, :, :, 0])
    xw_n = lax.bitcast_convert_type(
        xT.astype(jnp.bfloat16).reshape(N, C // 2, 2), jnp.int32
    )
    xw = jnp.transpose(xw_n).reshape(CGRP, pairs * N)

    ei = edge_index.reshape(2, N, K)
    eip = jnp.pad(ei, ((0, 0), (0, n_pad - N), (0, 0)))
    idxb = eip.reshape(2, nblk, CH, K).transpose(0, 1, 3, 2)

    m = _sc_maxdiff(xw, idxb, N, n_pad, K)

    We = W[:, 0::2]
    Wo = W[:, 1::2]
    part = _tc_partial(xf, We, b.reshape(C, 1), C, N)
    y = _tc_conv(part, m, Wo, C, N)
    return y.reshape(x.shape)


# speedup vs baseline: 1.0054x; 1.0054x over previous
import functools

import jax
import jax.numpy as jnp
from jax import lax
from jax.experimental import pallas as pl
from jax.experimental.pallas import tpu as pltpu
from jax.experimental.pallas import tpu_sc as plsc

NTILES = 32
NGRP = 2
CGRP = 16
CH = 256
LANES = 16


def _sc_maxdiff(xw, idxb, n_tab, n_pad, k_deg):
    pairs = xw.shape[1] // n_tab
    n_sub = n_pad // NGRP
    blocks = n_sub // CH
    groups = CH // LANES
    mesh = plsc.VectorSubcoreMesh(core_axis_name="c", subcore_axis_name="s")

    @functools.partial(
        pl.kernel,
        out_type=jax.ShapeDtypeStruct((2 * CGRP * pairs, n_pad), jnp.float32),
        mesh=mesh,
        compiler_params=pltpu.CompilerParams(
            needs_layout_passes=False,
            use_tc_tiling_on_sc=False,
        ),
        scratch_types=[
            pltpu.VMEM((pairs * n_tab,), jnp.int32),
            pltpu.VMEM((2 * pairs, n_sub), jnp.float32),
            pltpu.VMEM((k_deg, CH), jnp.int32),
            pltpu.VMEM((k_deg, CH), jnp.int32),
            pltpu.VMEM((k_deg, CH), jnp.int32),
            pltpu.VMEM((k_deg, CH), jnp.int32),
            pltpu.SemaphoreType.DMA,
            pltpu.SemaphoreType.DMA,
        ],
    )
    def sc_kernel(x_hbm, idx_hbm, m_hbm, table_v, out_v,
                  i0a, i1a, i0b, i1b, sem_a, sem_b):
        wid = lax.axis_index("s") * 2 + lax.axis_index("c")
        cg = wid // NGRP
        ng = wid % NGRP
        poffs = [jnp.full((LANES,), p * n_tab, jnp.int32) for p in range(pairs)]
        blk0 = ng * blocks

        def issue(blk, d0, d1, sem):
            pltpu.async_copy(idx_hbm.at[0, blk], d0, sem)
            pltpu.async_copy(idx_hbm.at[1, blk], d1, sem)

        def drain(blk, d0, d1, sem):
            pltpu.make_async_copy(idx_hbm.at[0, blk], d0, sem).wait()
            pltpu.make_async_copy(idx_hbm.at[1, blk], d1, sem).wait()

        def compute(j, b0, b1):
            def g_body(g, carry):
                base = g * LANES
                accs = []
                for kk in range(k_deg):
                    i0 = b0[kk, pl.ds(base, LANES)]
                    i1 = b1[kk, pl.ds(base, LANES)]
                    for p in range(pairs):
                        a = plsc.load_gather(table_v, [i0 + poffs[p]])
                        b2 = plsc.load_gather(table_v, [i1 + poffs[p]])
                        d = plsc.bitcast(a, jnp.bfloat16) - plsc.bitcast(b2, jnp.bfloat16)
                        if kk == 0:
                            accs.append(d)
                        else:
                            accs[p] = jnp.maximum(accs[p], d)
                loc = j * CH + base
                for p in range(pairs):
                    lo, hi = plsc.unpack(accs[p], format=plsc.PackFormat.INTERLEAVED)
                    out_v[2 * p, pl.ds(loc, LANES)] = lo
                    out_v[2 * p + 1, pl.ds(loc, LANES)] = hi
                return carry

            lax.fori_loop(0, groups, g_body, 0)

        issue(blk0, i0a, i1a, sem_a)
        pltpu.sync_copy(x_hbm.at[cg], table_v)

        def super_body(it, carry):
            ja = 2 * it
            jb = 2 * it + 1
            issue(blk0 + jb, i0b, i1b, sem_b)
            drain(blk0 + ja, i0a, i1a, sem_a)
            compute(ja, i0a, i1a)

            @pl.when(it + 1 < blocks // 2)
            def _():
                issue(blk0 + ja + 2, i0a, i1a, sem_a)

            drain(blk0 + jb, i0b, i1b, sem_b)
            compute(jb, i0b, i1b)
            return carry

        lax.fori_loop(0, blocks // 2, super_body, 0)
        pltpu.sync_copy(
            out_v,
            m_hbm.at[pl.ds(cg * 2 * pairs, 2 * pairs), pl.ds(ng * n_sub, n_sub)],
        )

    return sc_kernel(xw, idxb)


def _tc_partial(xf, We, b2, c, n):
    def body(x_ref, we_ref, b_ref, y_ref):
        y_ref[...] = (
            jnp.dot(we_ref[...], x_ref[...], preferred_element_type=jnp.float32)
            + b_ref[...]
        )

    return pl.pallas_call(
        body,
        out_shape=jax.ShapeDtypeStruct((c, n), jnp.float32),
    )(xf, We, b2)


def _tc_conv(part, m, Wo, c, n):
    def body(p_ref, m_ref, wo_ref, y_ref):
        acc = p_ref[...] + jnp.dot(
            wo_ref[...], m_ref[:, :n], preferred_element_type=jnp.float32
        )
        y_ref[...] = jnp.maximum(acc, 0.0)

    return pl.pallas_call(
        body,
        out_shape=jax.ShapeDtypeStruct((c, n), jnp.float32),
    )(part, m, Wo)


def kernel(x, edge_index, W, b):
    B, C, N, _ = x.shape
    K = edge_index.shape[-1]
    n_pad = ((N + (NGRP * CH) - 1) // (NGRP * CH)) * (NGRP * CH)
    nblk = n_pad // CH
    pairs = C // (2 * CGRP)

    xf = x.reshape(C, N)
    xb = xf.astype(jnp.bfloat16)
    xwords = lax.bitcast_convert_type(
        xb.reshape(C // 2, 2, N).transpose(0, 2, 1), jnp.int32
    )
    xw = xwords.reshape(CGRP, pairs * N)

    ei = edge_index.reshape(2, N, K)
    eip = jnp.pad(ei, ((0, 0), (0, n_pad - N), (0, 0)))
    idxb = eip.reshape(2, nblk, CH, K).transpose(0, 1, 3, 2)

    m = _sc_maxdiff(xw, idxb, N, n_pad, K)

    We = W[:, 0::2]
    Wo = W[:, 1::2]
    part = _tc_partial(xf, We, b.reshape(C, 1), C, N)
    y = _tc_conv(part, m, Wo, C, N)
    return y.reshape(x.shape)
